# baseline (speedup 1.0000x reference)
import jax
import jax.numpy as jnp
from jax.experimental import pallas as pl


def _bcast_add_kernel(w_ref, x_ref, o_ref):
    w = w_ref[...]
    x = x_ref[...]
    o_ref[...] = w[:, None, :] + x[None, :, :]


def kernel(x, pos_embed_weight):
    seq_len, batch_size = x.shape
    _, dim = pos_embed_weight.shape

    w = pos_embed_weight[:seq_len]

    BI = 8
    out = pl.pallas_call(
        _bcast_add_kernel,
        grid=(seq_len // BI,),
        in_specs=[
            pl.BlockSpec((BI, dim), lambda i: (i, 0)),
            pl.BlockSpec((seq_len, 1), lambda i: (0, 0)),
        ],
        out_specs=pl.BlockSpec((BI, seq_len, dim), lambda i: (i, 0, 0)),
        out_shape=jax.ShapeDtypeStruct((seq_len, seq_len, dim), jnp.float32),
    )(w, x)

    return out


# speedup vs baseline: 1.0588x; 1.0588x over previous
import jax
import jax.numpy as jnp
from jax import lax
from jax.experimental import pallas as pl
from jax.experimental.pallas import tpu as pltpu
from jax.experimental.pallas import tpu_sc as plsc

_S = 2048
_D = 16
_NW = 32
_RPW = _S // _NW
_CHUNK = 512
_NCH = _S // _CHUNK
_UNROLL = 16


def _sc_body(xs_hbm, w_hbm, out_hbm, xs_v, w_v, ob0, ob1, sem0, sem1):
    c = lax.axis_index("c")
    s = lax.axis_index("s")
    wid = s * 2 + c
    base = wid * _RPW

    pltpu.sync_copy(xs_hbm, xs_v)
    pltpu.sync_copy(w_hbm.at[pl.ds(base, _RPW)], w_v)

    bufs = (ob0, ob1)
    sems = (sem0, sem1)

    def compute_chunk(r, ci, ob):
        wrow = w_v[r, :]

        def inner(j0, carry):
            for u in range(_UNROLL):
                j = j0 * _UNROLL + u
                ob[j, :] = xs_v[ci * _CHUNK + j, :] + wrow
            return carry

        lax.fori_loop(0, _CHUNK // _UNROLL, inner, 0)

    def row(r, carry):
        for ci in range(_NCH):
            ob = bufs[ci % 2]
            sem = sems[ci % 2]
            dst = out_hbm.at[base + r, pl.ds(ci * _CHUNK, _CHUNK)]
            if ci >= 2:
                pltpu.make_async_copy(ob, dst, sem).wait()
            else:
                @pl.when(r > 0)
                def _(ob=ob, dst=dst, sem=sem):
                    pltpu.make_async_copy(ob, dst, sem).wait()
            compute_chunk(r, ci, ob)
            pltpu.async_copy(ob, dst, sem)
        return carry

    lax.fori_loop(0, _RPW, row, 0)

    drain = out_hbm.at[base, pl.ds(0, _CHUNK)]
    pltpu.make_async_copy(ob0, drain, sem0).wait()
    pltpu.make_async_copy(ob1, drain, sem1).wait()


def kernel(x, pos_embed_weight):
    seq_len, batch_size = x.shape
    _, dim = pos_embed_weight.shape

    xs = jnp.broadcast_to(x[:, :1], (seq_len, dim))
    w = pos_embed_weight[:seq_len]

    run = pl.kernel(
        _sc_body,
        out_type=jax.ShapeDtypeStruct((seq_len, seq_len, dim), jnp.float32),
        mesh=plsc.VectorSubcoreMesh(core_axis_name="c", subcore_axis_name="s"),
        scratch_types=[
            pltpu.VMEM((seq_len, dim), jnp.float32),
            pltpu.VMEM((_RPW, dim), jnp.float32),
            pltpu.VMEM((_CHUNK, dim), jnp.float32),
            pltpu.VMEM((_CHUNK, dim), jnp.float32),
            pltpu.SemaphoreType.DMA,
            pltpu.SemaphoreType.DMA,
        ],
        compiler_params=pltpu.CompilerParams(use_tc_tiling_on_sc=False),
    )
    return run(xs, w)


# speedup vs baseline: 20.6008x; 19.4566x over previous
import jax
import jax.numpy as jnp
from jax.experimental import pallas as pl


def _bcast_add_kernel(w_ref, xt_ref, o_ref):
    w = w_ref[...]
    xt = xt_ref[...]
    o_ref[...] = w[:, :, None] + xt[None, :, :]


def kernel(x, pos_embed_weight):
    seq_len, batch_size = x.shape
    _, dim = pos_embed_weight.shape

    w = pos_embed_weight[:seq_len]
    xt = x.reshape(1, seq_len)

    BI = 64
    out3 = pl.pallas_call(
        _bcast_add_kernel,
        grid=(seq_len // BI,),
        in_specs=[
            pl.BlockSpec((BI, dim), lambda i: (i, 0)),
            pl.BlockSpec((1, seq_len), lambda i: (0, 0)),
        ],
        out_specs=pl.BlockSpec((BI, dim, seq_len), lambda i: (i, 0, 0)),
        out_shape=jax.ShapeDtypeStruct((seq_len, dim, seq_len), jnp.float32),
    )(w, xt)

    return jnp.transpose(out3, (0, 2, 1))
